# Initial kernel scaffold; baseline (speedup 1.0000x reference)
import jax
import jax.numpy as jnp
from jax.experimental import pallas as pl


def kernel(x, edge_index, edge_attr, batch, We1, be1, Wr1, b1, We2, be2, Wr2, b2, We3, be3, Wr3, b3, bn1_g, bn1_b, bn2_g, bn2_b, mw1, mb1, mw2, mb2):
    raise NotImplementedError("write your pallas kernel here")


# speedup vs baseline: 1.9881x; 1.9881x over previous
import functools

import jax
import jax.numpy as jnp
from jax import lax
from jax.experimental import pallas as pl
from jax.experimental.pallas import tpu as pltpu
from jax.experimental.pallas import tpu_sc as plsc

N = 10000
E = 160000
F_IN = 128
D_E = 16
H = 16
T = 10
G = 64
EPS = 1e-5

NC = 2
NS = 16
NW = NC * NS
BE = 128
NB = 40
E_PAD = NW * NB * BE
DCH = D_E + 1
UW = DCH * H
N_PAD = 10240
RPT = N_PAD // NS

BN = 400
NBLK = N // BN


def _bcast_lane(v, d):
    idx = jnp.full((H,), d, dtype=jnp.int32)
    return lax.gather(
        v, idx[:, None],
        dimension_numbers=lax.GatherDimensionNumbers(
            offset_dims=(), collapsed_slice_dims=(0,), start_index_map=(0,)),
        slice_sizes=(1,), mode=lax.GatherScatterMode.PROMISE_IN_BOUNDS)


def _edge_body(u_hbm, ea_hbm, src_hbm, dst_hbm, zeros_hbm, out_hbm,
               src_v, dst_v, ea_v, rows_v, msg_v, agg_sh, sem):
    cid = lax.axis_index("c")
    sid = lax.axis_index("s")
    wid = sid * NC + cid

    pltpu.sync_copy(zeros_hbm.at[pl.ds(sid * RPT, RPT)],
                    agg_sh.at[pl.ds(sid * RPT, RPT)])
    plsc.subcore_barrier()

    def batch_body(j, carry):
        pltpu.sync_copy(src_hbm.at[wid, j], src_v)
        pltpu.sync_copy(dst_hbm.at[wid, j], dst_v)
        pltpu.sync_copy(ea_hbm.at[wid, j], ea_v)
        pltpu.async_copy(u_hbm.at[src_v], rows_v, sem).wait()

        def edge_body(e, c):
            ea0 = ea_v[e, pl.ds(0, H)]
            ea1 = ea_v[e, pl.ds(H, H)]
            acc = _bcast_lane(ea1, 0) * rows_v[e, pl.ds(D_E * H, H)]
            for dd in range(D_E):
                acc = acc + _bcast_lane(ea0, dd) * rows_v[e, pl.ds(dd * H, H)]
            msg_v[e, :] = acc
            return c

        lax.fori_loop(0, BE, edge_body, 0, unroll=2)
        pltpu.sync_copy(msg_v, agg_sh.at[dst_v], add=True)
        return carry

    lax.fori_loop(0, NB, batch_body, 0)
    plsc.subcore_barrier()
    pltpu.sync_copy(agg_sh.at[pl.ds(sid * RPT, RPT)],
                    out_hbm.at[cid, pl.ds(sid * RPT, RPT)])


@functools.lru_cache(maxsize=1)
def _make_edge_pass():
    return functools.partial(
        pl.kernel,
        out_type=jax.ShapeDtypeStruct((NC, N_PAD, H), jnp.float32),
        mesh=plsc.VectorSubcoreMesh(core_axis_name="c", subcore_axis_name="s",
                                    num_cores=NC),
        scratch_types=[
            pltpu.VMEM((BE,), jnp.int32),
            pltpu.VMEM((BE,), jnp.int32),
            pltpu.VMEM((BE, 2 * H), jnp.float32),
            pltpu.VMEM((BE, UW), jnp.float32),
            pltpu.VMEM((BE, H), jnp.float32),
            pltpu.VMEM_SHARED((N_PAD, H), jnp.float32),
            pltpu.SemaphoreType.DMA,
        ],
        compiler_params=pltpu.CompilerParams(use_tc_tiling_on_sc=False),
    )(_edge_body)


def _pre_body(x_ref, k_ref, wr_ref, b_ref, batch_ref, u_ref, xr_ref, cnt_ref):
    i = pl.program_id(0)
    xb = x_ref[...]
    u_ref[...] = jnp.dot(xb, k_ref[...], preferred_element_type=jnp.float32)
    xr_ref[...] = jnp.dot(xb, wr_ref[...],
                          preferred_element_type=jnp.float32) + b_ref[...]
    bt = batch_ref[0, 0, :]
    onehot = (bt[:, None] == lax.broadcasted_iota(jnp.int32, (1, G), 1)
              ).astype(jnp.float32)
    part = jnp.sum(onehot, axis=0, keepdims=True)

    @pl.when(i == 0)
    def _():
        cnt_ref[...] = jnp.zeros_like(cnt_ref)

    cnt_ref[...] += part


def _mid_body(agg_a_ref, agg_b_ref, xr_ref, batch_ref, cnt_ref, c_ref, bb_ref,
              k_ref, wr_ref, rb_ref, u_ref, xrn_ref):
    agg = agg_a_ref[0] + agg_b_ref[0] + xr_ref[...]
    cnt = cnt_ref[...]
    inv = jnp.where(cnt > 0.0, lax.rsqrt(jnp.maximum(cnt, 1.0)), 0.0)
    bt = batch_ref[0, 0, :]
    onehot = (bt[:, None] == lax.broadcasted_iota(jnp.int32, (1, G), 1)
              ).astype(jnp.float32)
    scale = jnp.sum(onehot * inv, axis=1, keepdims=True)
    h = jnp.maximum(agg * scale * c_ref[...] + bb_ref[...], 0.0)
    u_ref[...] = jnp.dot(h, k_ref[...], preferred_element_type=jnp.float32)
    xrn_ref[...] = jnp.dot(h, wr_ref[...],
                           preferred_element_type=jnp.float32) + rb_ref[...]


def _final_body(agg_a_ref, agg_b_ref, xr_ref, batch_ref, cnt_ref,
                mw1_ref, mb1_ref, mw2_ref, mb2_ref, out_ref, acc_ref):
    i = pl.program_id(0)
    h3 = agg_a_ref[0] + agg_b_ref[0] + xr_ref[...]
    bt = batch_ref[0, 0, :]
    onehot = (lax.broadcasted_iota(jnp.int32, (G, BN), 0) == bt[None, :]
              ).astype(jnp.float32)
    part = jnp.dot(onehot, h3, preferred_element_type=jnp.float32)

    @pl.when(i == 0)
    def _():
        acc_ref[...] = jnp.zeros_like(acc_ref)

    acc_ref[...] += part

    @pl.when(i == NBLK - 1)
    def _():
        pooled = acc_ref[...] / jnp.maximum(cnt_ref[...], 1.0)
        z = jnp.maximum(jnp.dot(pooled, mw1_ref[...],
                                preferred_element_type=jnp.float32)
                        + mb1_ref[...], 0.0)
        out_ref[...] = jnp.dot(z, mw2_ref[...],
                               preferred_element_type=jnp.float32) + mb2_ref[...]


def _full(shape):
    return pl.BlockSpec(shape, lambda i: tuple(0 for _ in shape))


def _pre_call(x, K, Wr, brow, batch3):
    return pl.pallas_call(
        _pre_body,
        grid=(NBLK,),
        in_specs=[
            pl.BlockSpec((BN, F_IN), lambda i: (i, 0)),
            _full((F_IN, UW)),
            _full((F_IN, H)),
            _full((1, H)),
            pl.BlockSpec((1, 1, BN), lambda i: (i, 0, 0)),
        ],
        out_specs=[
            pl.BlockSpec((BN, UW), lambda i: (i, 0)),
            pl.BlockSpec((BN, H), lambda i: (i, 0)),
            _full((1, G)),
        ],
        out_shape=[
            jax.ShapeDtypeStruct((N, UW), jnp.float32),
            jax.ShapeDtypeStruct((N, H), jnp.float32),
            jax.ShapeDtypeStruct((1, G), jnp.float32),
        ],
        compiler_params=pltpu.CompilerParams(
            dimension_semantics=("arbitrary",)),
    )(x, K, Wr, brow, batch3)


def _mid_call(aggs, xr, batch3, cnt, c, bb, K, Wr, rbrow):
    return pl.pallas_call(
        _mid_body,
        grid=(NBLK,),
        in_specs=[
            pl.BlockSpec((1, BN, H), lambda i: (0, i, 0)),
            pl.BlockSpec((1, BN, H), lambda i: (1, i, 0)),
            pl.BlockSpec((BN, H), lambda i: (i, 0)),
            pl.BlockSpec((1, 1, BN), lambda i: (i, 0, 0)),
            _full((1, G)),
            _full((1, H)),
            _full((1, H)),
            _full((H, UW)),
            _full((H, H)),
            _full((1, H)),
        ],
        out_specs=[
            pl.BlockSpec((BN, UW), lambda i: (i, 0)),
            pl.BlockSpec((BN, H), lambda i: (i, 0)),
        ],
        out_shape=[
            jax.ShapeDtypeStruct((N, UW), jnp.float32),
            jax.ShapeDtypeStruct((N, H), jnp.float32),
        ],
        compiler_params=pltpu.CompilerParams(
            dimension_semantics=("arbitrary",)),
    )(aggs, aggs, xr, batch3, cnt, c, bb, K, Wr, rbrow)


def _final_call(aggs, xr, batch3, cntcol, mw1, mb1row, mw2, mb2row):
    return pl.pallas_call(
        _final_body,
        grid=(NBLK,),
        in_specs=[
            pl.BlockSpec((1, BN, H), lambda i: (0, i, 0)),
            pl.BlockSpec((1, BN, H), lambda i: (1, i, 0)),
            pl.BlockSpec((BN, H), lambda i: (i, 0)),
            pl.BlockSpec((1, 1, BN), lambda i: (i, 0, 0)),
            _full((G, 1)),
            _full((H, H)),
            _full((1, H)),
            _full((H, T)),
            _full((1, T)),
        ],
        out_specs=pl.BlockSpec((G, T), lambda i: (0, 0)),
        out_shape=jax.ShapeDtypeStruct((G, T), jnp.float32),
        scratch_shapes=[pltpu.VMEM((G, H), jnp.float32)],
        compiler_params=pltpu.CompilerParams(
            dimension_semantics=("arbitrary",)),
    )(aggs, aggs, xr, batch3, cntcol, mw1, mb1row, mw2, mb2row)


def _mk_K(We, be, in_ch):
    Wer = We.reshape(D_E, in_ch, H).transpose(1, 0, 2).reshape(in_ch, D_E * H)
    return jnp.concatenate([Wer, be.reshape(in_ch, H)], axis=1)


def kernel(x, edge_index, edge_attr, batch, We1, be1, Wr1, b1, We2, be2, Wr2,
           b2, We3, be3, Wr3, b3, bn1_g, bn1_b, bn2_g, bn2_b, mw1, mb1, mw2,
           mb2):
    src = edge_index[0]
    dst = edge_index[1]

    K1 = _mk_K(We1, be1, F_IN)
    K2 = _mk_K(We2, be2, H)
    K3 = _mk_K(We3, be3, H)

    pad = E_PAD - E
    ea_aug = jnp.concatenate(
        [edge_attr, jnp.ones((E, 1), jnp.float32),
         jnp.zeros((E, 2 * H - DCH), jnp.float32)], axis=1)
    ea_p = jnp.pad(ea_aug, ((0, pad), (0, 0))).reshape(NW, NB, BE, 2 * H)
    src_p = jnp.pad(src, (0, pad)).reshape(NW, NB, BE)
    dst_p = jnp.pad(dst, (0, pad)).reshape(NW, NB, BE)
    zeros_n = jnp.zeros((N_PAD, H), jnp.float32)
    batch3 = batch.reshape(NBLK, 1, BN)

    bnc = 1.0 / (1.0 + EPS) ** 0.5
    c1 = (bn1_g * bnc).reshape(1, H)
    c2 = (bn2_g * bnc).reshape(1, H)

    U1, xr1, cnt = _pre_call(x, K1, Wr1, b1.reshape(1, H), batch3)
    aggs1 = _make_edge_pass()(U1, ea_p, src_p, dst_p, zeros_n)
    U2, xr2 = _mid_call(aggs1, xr1, batch3, cnt, c1, bn1_b.reshape(1, H),
                        K2, Wr2, b2.reshape(1, H))
    aggs2 = _make_edge_pass()(U2, ea_p, src_p, dst_p, zeros_n)
    U3, xr3 = _mid_call(aggs2, xr2, batch3, cnt, c2, bn2_b.reshape(1, H),
                        K3, Wr3, b3.reshape(1, H))
    aggs3 = _make_edge_pass()(U3, ea_p, src_p, dst_p, zeros_n)
    out = _final_call(aggs3, xr3, batch3, cnt.reshape(G, 1), mw1,
                      mb1.reshape(1, H), mw2, mb2.reshape(1, T))
    return out
